# baseline (speedup 1.0000x reference)
import functools

import jax
import jax.numpy as jnp
from jax import lax
from jax.experimental import pallas as pl
from jax.experimental.pallas import tpu as pltpu
from jax.experimental.pallas import tpu_sc as plsc

N = 10000
E = 320000
F = 128
H = 128
NC = 2
NS = 16
NW = NC * NS
NP = 10240
DUMMY = 10200
EPT = E // NW
NB = (EPT + 127) // 128
EPS = NB * 128
TAIL = EPT - (NB - 1) * 128
TPAD = EPS - EPT
SLICE = NP // NS

_MESH = plsc.VectorSubcoreMesh(core_axis_name="c", subcore_axis_name="s")


def _newton_rsqrt(x):
    y = 1.0 / x
    for _ in range(20):
        y = y * (1.5 - 0.5 * x * y * y)
    return y


@functools.partial(
    pl.kernel,
    out_type=(
        jax.ShapeDtypeStruct((NC * NP,), jnp.float32),
        jax.ShapeDtypeStruct((NC * NP,), jnp.float32),
    ),
    mesh=_MESH,
    scratch_types=(
        pltpu.VMEM((EPS,), jnp.int32),
        pltpu.VMEM((EPS,), jnp.int32),
        pltpu.VMEM((NB, 128), jnp.int32),
        pltpu.VMEM((EPS,), jnp.float32),
        pltpu.VMEM((128,), jnp.float32),
        pltpu.VMEM((SLICE,), jnp.float32),
        pltpu.VMEM_SHARED((NP,), jnp.float32),
        pltpu.VMEM_SHARED((NP,), jnp.float32),
    ),
)
def _count_kernel(edge_hbm, cnt_out, self_out, rowst, colst, col2d, vals,
                  ones, vbuf, cnt_sp, self_sp):
    c = lax.axis_index("c")
    s = lax.axis_index("s")
    wid = c * NS + s
    base = wid * EPT

    def zb(i, _):
        vbuf[pl.ds(i * 16, 16)] = jnp.zeros((16,), jnp.float32)
        return 0
    lax.fori_loop(0, SLICE // 16, zb, 0)
    pltpu.sync_copy(vbuf, cnt_sp.at[pl.ds(s * SLICE, SLICE)])
    pltpu.sync_copy(vbuf, self_sp.at[pl.ds(s * SLICE, SLICE)])

    pltpu.sync_copy(edge_hbm.at[pl.ds(base, EPT)], rowst.at[pl.ds(0, EPT)])
    pltpu.sync_copy(edge_hbm.at[pl.ds(E + base, EPT)], colst.at[pl.ds(0, EPT)])
    for k in range(TPAD // 16):
        rowst[pl.ds(EPT + k * 16, 16)] = jnp.zeros((16,), jnp.int32)
        colst[pl.ds(EPT + k * 16, 16)] = jnp.full((16,), DUMMY, jnp.int32)
    for k in range(8):
        ones[pl.ds(k * 16, 16)] = jnp.ones((16,), jnp.float32)

    def repack(j, _):
        for k in range(8):
            r16 = rowst[pl.ds(j * 128 + k * 16, 16)]
            c16 = colst[pl.ds(j * 128 + k * 16, 16)]
            vals[pl.ds(j * 128 + k * 16, 16)] = jnp.where(
                r16 == c16, 1.0, 0.0).astype(jnp.float32)
            col2d[j, pl.ds(k * 16, 16)] = c16
        return 0
    lax.fori_loop(0, NB, repack, 0)

    plsc.subcore_barrier()

    def scat(j, _):
        pltpu.sync_copy(ones, cnt_sp.at[col2d.at[j]], add=True)
        pltpu.sync_copy(vals.at[pl.ds(j * 128, 128)],
                        self_sp.at[col2d.at[j]], add=True)
        return 0
    lax.fori_loop(0, NB, scat, 0)

    plsc.subcore_barrier()

    pltpu.sync_copy(cnt_sp.at[pl.ds(s * SLICE, SLICE)], vbuf)
    pltpu.sync_copy(vbuf, cnt_out.at[pl.ds(c * NP + s * SLICE, SLICE)])
    pltpu.sync_copy(self_sp.at[pl.ds(s * SLICE, SLICE)], vbuf)
    pltpu.sync_copy(vbuf, self_out.at[pl.ds(c * NP + s * SLICE, SLICE)])


ROWS_A2 = NP // NW


@functools.partial(
    pl.kernel,
    out_type=(
        jax.ShapeDtypeStruct((NP,), jnp.float32),
        jax.ShapeDtypeStruct((NP,), jnp.float32),
    ),
    mesh=_MESH,
    scratch_types=(
        pltpu.VMEM((ROWS_A2,), jnp.float32),
        pltpu.VMEM((ROWS_A2,), jnp.float32),
        pltpu.VMEM((ROWS_A2,), jnp.float32),
        pltpu.VMEM((ROWS_A2,), jnp.float32),
        pltpu.VMEM((ROWS_A2,), jnp.float32),
    ),
)
def _coef_kernel(cnt_parts, self_parts, dis_out, gm_out, p, q, t, dis_v, gm_v):
    c = lax.axis_index("c")
    s = lax.axis_index("s")
    wid = c * NS + s
    base = wid * ROWS_A2

    pltpu.sync_copy(cnt_parts.at[pl.ds(base, ROWS_A2)], p)
    pltpu.sync_copy(cnt_parts.at[pl.ds(NP + base, ROWS_A2)], t)

    def addp(i, _):
        p[pl.ds(i * 16, 16)] = p[pl.ds(i * 16, 16)] + t[pl.ds(i * 16, 16)]
        return 0
    lax.fori_loop(0, ROWS_A2 // 16, addp, 0)

    pltpu.sync_copy(self_parts.at[pl.ds(base, ROWS_A2)], q)
    pltpu.sync_copy(self_parts.at[pl.ds(NP + base, ROWS_A2)], t)

    def addq(i, _):
        q[pl.ds(i * 16, 16)] = q[pl.ds(i * 16, 16)] + t[pl.ds(i * 16, 16)]
        return 0
    lax.fori_loop(0, ROWS_A2 // 16, addq, 0)

    def finalize(i, _):
        cnt16 = p[pl.ds(i * 16, 16)]
        self16 = q[pl.ds(i * 16, 16)]
        lm = jnp.where(self16 == 0.0, 1.0, 0.0).astype(jnp.float32)
        deg = cnt16 + lm
        dis = _newton_rsqrt(deg)
        dis = jnp.where(deg > 0.0, dis, 0.0).astype(jnp.float32)
        dis_v[pl.ds(i * 16, 16)] = dis
        gm_v[pl.ds(i * 16, 16)] = dis * lm
        return 0
    lax.fori_loop(0, ROWS_A2 // 16, finalize, 0)

    pltpu.sync_copy(dis_v, dis_out.at[pl.ds(base, ROWS_A2)])
    pltpu.sync_copy(gm_v, gm_out.at[pl.ds(base, ROWS_A2)])


BR = 2000


def _matmul_body(x_ref, w_ref, d_ref, o_ref):
    h = lax.dot_general(x_ref[...], w_ref[...], (((1,), (1,)), ((), ())),
                        preferred_element_type=jnp.float32)
    o_ref[...] = h * d_ref[...]


def _matmul(x, w, dis1):
    return pl.pallas_call(
        _matmul_body,
        grid=(N // BR,),
        in_specs=[
            pl.BlockSpec((BR, F), lambda i: (i, 0)),
            pl.BlockSpec((H, F), lambda i: (0, 0)),
            pl.BlockSpec((BR, 1), lambda i: (i, 0)),
        ],
        out_specs=pl.BlockSpec((BR, H), lambda i: (i, 0)),
        out_shape=jax.ShapeDtypeStruct((N, H), jnp.float32),
    )(x, w, dis1)


NPASS = 2
EPP = EPT // NPASS
NBP = (EPP + 127) // 128
EPSP = NBP * 128
TAILP = EPP - (NBP - 1) * 128


@functools.partial(
    pl.kernel,
    out_type=jax.ShapeDtypeStruct((NC, NP, H), jnp.float32),
    mesh=_MESH,
    scratch_types=(
        pltpu.VMEM((EPSP,), jnp.int32),
        pltpu.VMEM((NBP, 128), jnp.int32),
        pltpu.VMEM((128, H), jnp.float32),
        pltpu.VMEM((128, H), jnp.float32),
        pltpu.VMEM_SHARED((NP, H), jnp.float32),
        pltpu.SemaphoreType.DMA,
        pltpu.SemaphoreType.DMA,
    ),
)
def _scatter_kernel(hp_hbm, edge_hbm, s_out, rowst, col2d, rows0, rows1,
                    s_sp, semg, semc):
    c = lax.axis_index("c")
    s = lax.axis_index("s")
    wid = c * NS + s
    base = wid * EPT

    def zrow(r, _):
        for k in range(8):
            rows0[r, pl.ds(k * 16, 16)] = jnp.zeros((16,), jnp.float32)
        return 0
    lax.fori_loop(0, 128, zrow, 0)

    def zslab(i, _):
        pltpu.sync_copy(rows0, s_sp.at[pl.ds(s * SLICE + i * 128, 128)])
        return 0
    lax.fori_loop(0, SLICE // 128, zslab, 0)

    plsc.subcore_barrier()

    bufs = (rows0, rows1)

    def gather(j, buf):
        return pltpu.async_copy(
            hp_hbm.at[rowst.at[pl.ds(j * 128, 128)]], buf, semg)

    def gwait(buf):
        pltpu.make_async_copy(
            hp_hbm.at[rowst.at[pl.ds(0, 128)]], buf, semg).wait()

    for p in range(NPASS):
        pbase = base + p * EPP
        for k in range(8):
            rowst[pl.ds(EPSP - 128 + k * 16, 16)] = jnp.zeros((16,), jnp.int32)
            col2d[NBP - 1, pl.ds(k * 16, 16)] = jnp.full(
                (16,), DUMMY, jnp.int32)
        descs = [pltpu.async_copy(
            edge_hbm.at[pl.ds(pbase, EPP)], rowst.at[pl.ds(0, EPP)], semc)]
        for j in range(NBP - 1):
            descs.append(pltpu.async_copy(
                edge_hbm.at[pl.ds(E + pbase + j * 128, 128)],
                col2d.at[j], semc))
        descs.append(pltpu.async_copy(
            edge_hbm.at[pl.ds(E + pbase + (NBP - 1) * 128, TAILP)],
            col2d.at[NBP - 1, pl.ds(0, TAILP)], semc))
        for d in descs:
            d.wait()

        gather(0, rows0)
        gather(1, rows1)

        def pair(i, _):
            j0 = 2 * i
            gwait(rows0)
            pltpu.sync_copy(rows0, s_sp.at[col2d.at[j0]], add=True)

            @pl.when(j0 + 2 < NBP)
            def _():
                gather(j0 + 2, rows0)

            gwait(rows1)
            pltpu.sync_copy(rows1, s_sp.at[col2d.at[j0 + 1]], add=True)

            @pl.when(j0 + 3 < NBP)
            def _():
                gather(j0 + 3, rows1)
            return 0
        lax.fori_loop(0, NBP // 2, pair, 0)

    plsc.subcore_barrier()

    def dump(i, _):
        sl = pl.ds(s * SLICE + i * 128, 128)
        pltpu.sync_copy(s_sp.at[sl], rows0)
        pltpu.sync_copy(rows0, s_out.at[c, sl])
        return 0
    lax.fori_loop(0, SLICE // 128, dump, 0)


def _combine_body(s_ref, d_ref, gm_ref, hp_ref, b_ref, o_ref):
    stot = s_ref[0] + s_ref[1]
    o_ref[...] = d_ref[...] * stot + gm_ref[...] * hp_ref[...] + b_ref[...]


def _combine(s_parts, dis1, gm1, hp, bias2d):
    return pl.pallas_call(
        _combine_body,
        grid=(N // BR,),
        in_specs=[
            pl.BlockSpec((NC, BR, H), lambda i: (0, i, 0)),
            pl.BlockSpec((BR, 1), lambda i: (i, 0)),
            pl.BlockSpec((BR, 1), lambda i: (i, 0)),
            pl.BlockSpec((BR, H), lambda i: (i, 0)),
            pl.BlockSpec((1, H), lambda i: (0, 0)),
        ],
        out_specs=pl.BlockSpec((BR, H), lambda i: (i, 0)),
        out_shape=jax.ShapeDtypeStruct((N, H), jnp.float32),
    )(s_parts, dis1, gm1, hp, bias2d)


def kernel(x, edge_index, adj_norm_sp, W, bias):
    del adj_norm_sp
    edge_flat = edge_index.astype(jnp.int32).reshape(2 * E)
    cnt_parts, self_parts = _count_kernel(edge_flat)
    dis, gm = _coef_kernel(cnt_parts, self_parts)
    dis1 = dis.reshape(NP, 1)
    gm1 = gm.reshape(NP, 1)
    hp = _matmul(x, W, dis1)
    s_parts = _scatter_kernel(hp, edge_flat)
    out = _combine(s_parts, dis1, gm1, hp, bias.reshape(1, H))
    return out


# speedup vs baseline: 1.2644x; 1.2644x over previous
import functools

import jax
import jax.numpy as jnp
from jax import lax
from jax.experimental import pallas as pl
from jax.experimental.pallas import tpu as pltpu
from jax.experimental.pallas import tpu_sc as plsc

N = 10000
E = 320000
F = 128
H = 128
NC = 2
NS = 16
NW = NC * NS
NP = 10240
DUMMY = 10200
EPT = E // NW
NB = (EPT + 127) // 128
EPS = NB * 128
TAIL = EPT - (NB - 1) * 128
TPAD = EPS - EPT
SLICE = NP // NS

_MESH = plsc.VectorSubcoreMesh(core_axis_name="c", subcore_axis_name="s")


def _newton_rsqrt(x):
    y = 1.0 / x
    for _ in range(20):
        y = y * (1.5 - 0.5 * x * y * y)
    return y


@functools.partial(
    pl.kernel,
    out_type=(
        jax.ShapeDtypeStruct((NC * NP,), jnp.float32),
        jax.ShapeDtypeStruct((NC * NP,), jnp.float32),
    ),
    mesh=_MESH,
    scratch_types=(
        pltpu.VMEM((EPS,), jnp.int32),
        pltpu.VMEM((EPS,), jnp.int32),
        pltpu.VMEM((NB, 128), jnp.int32),
        pltpu.VMEM((EPS,), jnp.float32),
        pltpu.VMEM((128,), jnp.float32),
        pltpu.VMEM((SLICE,), jnp.float32),
        pltpu.VMEM_SHARED((NP,), jnp.float32),
        pltpu.VMEM_SHARED((NP,), jnp.float32),
        pltpu.SemaphoreType.DMA,
    ),
)
def _count_kernel(edge_hbm, cnt_out, self_out, rowst, colst, col2d, vals,
                  ones, vbuf, cnt_sp, self_sp, scsem):
    c = lax.axis_index("c")
    s = lax.axis_index("s")
    wid = c * NS + s
    base = wid * EPT

    def zb(i, _):
        vbuf[pl.ds(i * 16, 16)] = jnp.zeros((16,), jnp.float32)
        return 0
    lax.fori_loop(0, SLICE // 16, zb, 0)
    pltpu.sync_copy(vbuf, cnt_sp.at[pl.ds(s * SLICE, SLICE)])
    pltpu.sync_copy(vbuf, self_sp.at[pl.ds(s * SLICE, SLICE)])

    pltpu.sync_copy(edge_hbm.at[pl.ds(base, EPT)], rowst.at[pl.ds(0, EPT)])
    pltpu.sync_copy(edge_hbm.at[pl.ds(E + base, EPT)], colst.at[pl.ds(0, EPT)])
    for k in range(TPAD // 16):
        rowst[pl.ds(EPT + k * 16, 16)] = jnp.zeros((16,), jnp.int32)
        colst[pl.ds(EPT + k * 16, 16)] = jnp.full((16,), DUMMY, jnp.int32)
    for k in range(8):
        ones[pl.ds(k * 16, 16)] = jnp.ones((16,), jnp.float32)

    def repack(j, _):
        for k in range(8):
            r16 = rowst[pl.ds(j * 128 + k * 16, 16)]
            c16 = colst[pl.ds(j * 128 + k * 16, 16)]
            vals[pl.ds(j * 128 + k * 16, 16)] = jnp.where(
                r16 == c16, 1.0, 0.0).astype(jnp.float32)
            col2d[j, pl.ds(k * 16, 16)] = c16
        return 0
    lax.fori_loop(0, NB, repack, 0)

    plsc.subcore_barrier()

    sdescs = []
    for j in range(NB):
        sdescs.append(pltpu.async_copy(
            ones, cnt_sp.at[col2d.at[j]], scsem, add=True))
        sdescs.append(pltpu.async_copy(
            vals.at[pl.ds(j * 128, 128)], self_sp.at[col2d.at[j]],
            scsem, add=True))
    for d in sdescs:
        d.wait()

    plsc.subcore_barrier()

    pltpu.sync_copy(cnt_sp.at[pl.ds(s * SLICE, SLICE)], vbuf)
    pltpu.sync_copy(vbuf, cnt_out.at[pl.ds(c * NP + s * SLICE, SLICE)])
    pltpu.sync_copy(self_sp.at[pl.ds(s * SLICE, SLICE)], vbuf)
    pltpu.sync_copy(vbuf, self_out.at[pl.ds(c * NP + s * SLICE, SLICE)])


ROWS_A2 = NP // NW


@functools.partial(
    pl.kernel,
    out_type=(
        jax.ShapeDtypeStruct((NP,), jnp.float32),
        jax.ShapeDtypeStruct((NP,), jnp.float32),
    ),
    mesh=_MESH,
    scratch_types=(
        pltpu.VMEM((ROWS_A2,), jnp.float32),
        pltpu.VMEM((ROWS_A2,), jnp.float32),
        pltpu.VMEM((ROWS_A2,), jnp.float32),
        pltpu.VMEM((ROWS_A2,), jnp.float32),
        pltpu.VMEM((ROWS_A2,), jnp.float32),
    ),
)
def _coef_kernel(cnt_parts, self_parts, dis_out, gm_out, p, q, t, dis_v, gm_v):
    c = lax.axis_index("c")
    s = lax.axis_index("s")
    wid = c * NS + s
    base = wid * ROWS_A2

    pltpu.sync_copy(cnt_parts.at[pl.ds(base, ROWS_A2)], p)
    pltpu.sync_copy(cnt_parts.at[pl.ds(NP + base, ROWS_A2)], t)

    def addp(i, _):
        p[pl.ds(i * 16, 16)] = p[pl.ds(i * 16, 16)] + t[pl.ds(i * 16, 16)]
        return 0
    lax.fori_loop(0, ROWS_A2 // 16, addp, 0)

    pltpu.sync_copy(self_parts.at[pl.ds(base, ROWS_A2)], q)
    pltpu.sync_copy(self_parts.at[pl.ds(NP + base, ROWS_A2)], t)

    def addq(i, _):
        q[pl.ds(i * 16, 16)] = q[pl.ds(i * 16, 16)] + t[pl.ds(i * 16, 16)]
        return 0
    lax.fori_loop(0, ROWS_A2 // 16, addq, 0)

    def finalize(i, _):
        cnt16 = p[pl.ds(i * 16, 16)]
        self16 = q[pl.ds(i * 16, 16)]
        lm = jnp.where(self16 == 0.0, 1.0, 0.0).astype(jnp.float32)
        deg = cnt16 + lm
        dis = _newton_rsqrt(deg)
        dis = jnp.where(deg > 0.0, dis, 0.0).astype(jnp.float32)
        dis_v[pl.ds(i * 16, 16)] = dis
        gm_v[pl.ds(i * 16, 16)] = dis * lm
        return 0
    lax.fori_loop(0, ROWS_A2 // 16, finalize, 0)

    pltpu.sync_copy(dis_v, dis_out.at[pl.ds(base, ROWS_A2)])
    pltpu.sync_copy(gm_v, gm_out.at[pl.ds(base, ROWS_A2)])


BR = 2000


def _matmul_body(x_ref, w_ref, d_ref, o_ref):
    h = lax.dot_general(x_ref[...], w_ref[...], (((1,), (1,)), ((), ())),
                        preferred_element_type=jnp.float32)
    o_ref[...] = h * d_ref[...]


def _matmul(x, w, dis1):
    return pl.pallas_call(
        _matmul_body,
        grid=(N // BR,),
        in_specs=[
            pl.BlockSpec((BR, F), lambda i: (i, 0)),
            pl.BlockSpec((H, F), lambda i: (0, 0)),
            pl.BlockSpec((BR, 1), lambda i: (i, 0)),
        ],
        out_specs=pl.BlockSpec((BR, H), lambda i: (i, 0)),
        out_shape=jax.ShapeDtypeStruct((N, H), jnp.float32),
    )(x, w, dis1)


@functools.partial(
    pl.kernel,
    out_type=jax.ShapeDtypeStruct((NC, NP, H), jnp.float32),
    mesh=_MESH,
    scratch_types=(
        pltpu.VMEM((EPS,), jnp.int32),
        pltpu.VMEM((NB, 128), jnp.int32),
        pltpu.VMEM((128, H), jnp.float32),
        pltpu.VMEM_SHARED((NP, H), jnp.float32),
        pltpu.SemaphoreType.DMA,
        pltpu.SemaphoreType.DMA,
    ),
)
def _scatter_kernel(hp_hbm, edge_hbm, s_out, rowst, col2d, rows_buf,
                    s_sp, sem, sem2):
    c = lax.axis_index("c")
    s = lax.axis_index("s")
    wid = c * NS + s
    base = wid * EPT

    descs = [pltpu.async_copy(
        edge_hbm.at[pl.ds(base, EPT)], rowst.at[pl.ds(0, EPT)], sem2)]
    for j in range(NB - 1):
        descs.append(pltpu.async_copy(
            edge_hbm.at[pl.ds(E + base + j * 128, 128)], col2d.at[j], sem2))
    descs.append(pltpu.async_copy(
        edge_hbm.at[pl.ds(E + base + (NB - 1) * 128, TAIL)],
        col2d.at[NB - 1, pl.ds(0, TAIL)], sem2))

    def zrow(r, _):
        for k in range(8):
            rows_buf[r, pl.ds(k * 16, 16)] = jnp.zeros((16,), jnp.float32)
        return 0
    lax.fori_loop(0, 128, zrow, 0)

    def zslab(i, _):
        pltpu.sync_copy(rows_buf, s_sp.at[pl.ds(s * SLICE + i * 128, 128)])
        return 0
    lax.fori_loop(0, SLICE // 128, zslab, 0)

    for d in descs:
        d.wait()
    for k in range(TPAD // 16):
        rowst[pl.ds(EPT + k * 16, 16)] = jnp.zeros((16,), jnp.int32)
        col2d[NB - 1, pl.ds(TAIL + k * 16, 16)] = jnp.full(
            (16,), DUMMY, jnp.int32)

    plsc.subcore_barrier()

    def step(j, _):
        pltpu.async_copy(
            hp_hbm.at[rowst.at[pl.ds(j * 128, 128)]], rows_buf, sem).wait()
        pltpu.sync_copy(rows_buf, s_sp.at[col2d.at[j]], add=True)
        return 0
    lax.fori_loop(0, NB, step, 0)

    plsc.subcore_barrier()

    def dump(i, _):
        sl = pl.ds(s * SLICE + i * 128, 128)
        pltpu.sync_copy(s_sp.at[sl], rows_buf)
        pltpu.sync_copy(rows_buf, s_out.at[c, sl])
        return 0
    lax.fori_loop(0, SLICE // 128, dump, 0)


def _combine_body(s_ref, d_ref, gm_ref, hp_ref, b_ref, o_ref):
    stot = s_ref[0] + s_ref[1]
    o_ref[...] = d_ref[...] * stot + gm_ref[...] * hp_ref[...] + b_ref[...]


def _combine(s_parts, dis1, gm1, hp, bias2d):
    return pl.pallas_call(
        _combine_body,
        grid=(N // BR,),
        in_specs=[
            pl.BlockSpec((NC, BR, H), lambda i: (0, i, 0)),
            pl.BlockSpec((BR, 1), lambda i: (i, 0)),
            pl.BlockSpec((BR, 1), lambda i: (i, 0)),
            pl.BlockSpec((BR, H), lambda i: (i, 0)),
            pl.BlockSpec((1, H), lambda i: (0, 0)),
        ],
        out_specs=pl.BlockSpec((BR, H), lambda i: (i, 0)),
        out_shape=jax.ShapeDtypeStruct((N, H), jnp.float32),
    )(s_parts, dis1, gm1, hp, bias2d)


def kernel(x, edge_index, adj_norm_sp, W, bias):
    del adj_norm_sp
    edge_flat = edge_index.astype(jnp.int32).reshape(2 * E)
    cnt_parts, self_parts = _count_kernel(edge_flat)
    dis, gm = _coef_kernel(cnt_parts, self_parts)
    dis1 = dis.reshape(NP, 1)
    gm1 = gm.reshape(NP, 1)
    hp = _matmul(x, W, dis1)
    s_parts = _scatter_kernel(hp, edge_flat)
    out = _combine(s_parts, dis1, gm1, hp, bias.reshape(1, H))
    return out
